# baseline (speedup 1.0000x reference)
import functools

import jax
import jax.numpy as jnp
from jax.experimental import pallas as pl
from jax.experimental.pallas import tpu as pltpu

_N_EXPERTS = 64
_TOP_K = 8
_CHUNK = 512
_DEPTH = 4


def _topk_chunk(x, w, idx_ref, wgt_ref, c):
    logits = jax.lax.dot_general(
        w, x, (((1,), (1,)), ((), ())), preferred_element_type=jnp.float32
    )
    t = logits.shape[1]
    rev = (
        jnp.int32(_N_EXPERTS - 1)
        - jax.lax.broadcasted_iota(jnp.int32, (_N_EXPERTS, t), 0)
    ).astype(jnp.float32)
    rm = jnp.max(logits, axis=0, keepdims=True)
    p = jnp.exp(logits - rm)
    vals = []
    lanes = []
    for _ in range(_TOP_K):
        m = jnp.max(p, axis=0, keepdims=True)
        r = jnp.max(jnp.where(p == m, rev, -1.0), axis=0, keepdims=True)
        vals.append(m)
        lanes.append(r)
        p = jnp.where(rev == r, -1.0, p)
    e = jnp.concatenate(vals, axis=0)
    r8 = jnp.concatenate(lanes, axis=0)
    idx = jnp.int32(_N_EXPERTS - 1) - r8.astype(jnp.int32)
    wgt = e / jnp.sum(e, axis=0, keepdims=True)
    idx_ref[pl.ds(c * _CHUNK, _CHUNK), :] = idx.T
    wgt_ref[pl.ds(c * _CHUNK, _CHUNK), :] = wgt.T


def _gate_kernel(x_hbm, w_ref, idx_ref, wgt_ref, bufs, sems, *, n_chunks):
    def copy(c):
        return pltpu.make_async_copy(
            x_hbm.at[pl.ds(c * _CHUNK, _CHUNK), :],
            bufs.at[c % _DEPTH],
            sems.at[c % _DEPTH],
        )

    for c in range(min(_DEPTH, n_chunks)):
        copy(c).start()
    for c in range(n_chunks):
        copy(c).wait()
        _topk_chunk(bufs[c % _DEPTH], w_ref[...], idx_ref, wgt_ref, c)
        if c + _DEPTH < n_chunks:
            copy(c + _DEPTH).start()


@functools.partial(jax.jit, static_argnums=())
def kernel(hidden_states, weight):
    bsz, seq, h = hidden_states.shape
    tokens = bsz * seq
    x = hidden_states.reshape(tokens, h).astype(jnp.float32)
    w = weight.astype(jnp.float32)
    n_chunks = tokens // _CHUNK
    idx, wgt = pl.pallas_call(
        functools.partial(_gate_kernel, n_chunks=n_chunks),
        in_specs=[
            pl.BlockSpec(memory_space=pltpu.MemorySpace.HBM),
            pl.BlockSpec(memory_space=pltpu.MemorySpace.VMEM),
        ],
        out_specs=[
            pl.BlockSpec(memory_space=pltpu.MemorySpace.VMEM),
            pl.BlockSpec(memory_space=pltpu.MemorySpace.VMEM),
        ],
        out_shape=[
            jax.ShapeDtypeStruct((tokens, _TOP_K), jnp.int32),
            jax.ShapeDtypeStruct((tokens, _TOP_K), jnp.float32),
        ],
        scratch_shapes=[
            pltpu.VMEM((_DEPTH, _CHUNK, h), jnp.float32),
            pltpu.SemaphoreType.DMA((_DEPTH,)),
        ],
    )(x, w)
    return idx, wgt


# speedup vs baseline: 1.0007x; 1.0007x over previous
import functools

import jax
import jax.numpy as jnp
from jax.experimental import pallas as pl
from jax.experimental.pallas import tpu as pltpu

_N_EXPERTS = 64
_TOP_K = 8
_CHUNK = 512
_DEPTH = 4
_NBUF = 6


def _topk_chunk(x, w, idx_ref, wgt_ref, c):
    logits = jax.lax.dot_general(
        w, x, (((1,), (1,)), ((), ())), preferred_element_type=jnp.float32
    )
    t = logits.shape[1]
    rev = (
        jnp.int32(_N_EXPERTS - 1)
        - jax.lax.broadcasted_iota(jnp.int32, (_N_EXPERTS, t), 0)
    ).astype(jnp.float32)
    rm = jnp.max(logits, axis=0, keepdims=True)
    p = jnp.exp(logits - rm)
    vals = []
    lanes = []
    for _ in range(_TOP_K):
        m = jnp.max(p, axis=0, keepdims=True)
        r = jnp.max(jnp.where(p == m, rev, -1.0), axis=0, keepdims=True)
        vals.append(m)
        lanes.append(r)
        p = jnp.where(rev == r, -1.0, p)
    e = jnp.concatenate(vals, axis=0)
    r8 = jnp.concatenate(lanes, axis=0)
    idx = jnp.int32(_N_EXPERTS - 1) - r8.astype(jnp.int32)
    wgt = e / jnp.sum(e, axis=0, keepdims=True)
    idx_ref[pl.ds(c * _CHUNK, _CHUNK), :] = idx.T
    wgt_ref[pl.ds(c * _CHUNK, _CHUNK), :] = wgt.T


def _gate_kernel(x_hbm, w_ref, idx_ref, wgt_ref, bufs, sems, *, n_chunks):
    def copy(c):
        return pltpu.make_async_copy(
            x_hbm.at[pl.ds(c * _CHUNK, _CHUNK), :],
            bufs.at[c % _NBUF],
            sems.at[c % _NBUF],
        )

    for c in range(min(_DEPTH, n_chunks)):
        copy(c).start()
    for c in range(n_chunks):
        copy(c).wait()
        if c + _DEPTH < n_chunks:
            copy(c + _DEPTH).start()
        _topk_chunk(bufs[c % _NBUF], w_ref[...], idx_ref, wgt_ref, c)


@functools.partial(jax.jit, static_argnums=())
def kernel(hidden_states, weight):
    bsz, seq, h = hidden_states.shape
    tokens = bsz * seq
    x = hidden_states.reshape(tokens, h).astype(jnp.float32)
    w = weight.astype(jnp.float32)
    n_chunks = tokens // _CHUNK
    idx, wgt = pl.pallas_call(
        functools.partial(_gate_kernel, n_chunks=n_chunks),
        in_specs=[
            pl.BlockSpec(memory_space=pltpu.MemorySpace.HBM),
            pl.BlockSpec(memory_space=pltpu.MemorySpace.VMEM),
        ],
        out_specs=[
            pl.BlockSpec(memory_space=pltpu.MemorySpace.VMEM),
            pl.BlockSpec(memory_space=pltpu.MemorySpace.VMEM),
        ],
        out_shape=[
            jax.ShapeDtypeStruct((tokens, _TOP_K), jnp.int32),
            jax.ShapeDtypeStruct((tokens, _TOP_K), jnp.float32),
        ],
        scratch_shapes=[
            pltpu.VMEM((_NBUF, _CHUNK, h), jnp.float32),
            pltpu.SemaphoreType.DMA((_NBUF,)),
        ],
    )(x, w)
    return idx, wgt


# speedup vs baseline: 1.0454x; 1.0447x over previous
import functools

import jax
import jax.numpy as jnp
from jax.experimental import pallas as pl
from jax.experimental.pallas import tpu as pltpu

_N_EXPERTS = 64
_TOP_K = 8
_TILE = 1024


def _gate_kernel(x_ref, w_ref, idx_ref, wgt_ref):
    x = x_ref[...]
    w = w_ref[...]
    logits = jax.lax.dot_general(
        w, x, (((1,), (1,)), ((), ())), preferred_element_type=jnp.float32
    )
    t = logits.shape[1]
    rev = (
        jnp.int32(_N_EXPERTS - 1)
        - jax.lax.broadcasted_iota(jnp.int32, (_N_EXPERTS, t), 0)
    ).astype(jnp.float32)
    neg = jnp.float32(-jnp.inf)
    p = logits
    vals = []
    lanes = []
    for _ in range(_TOP_K):
        m = jnp.max(p, axis=0, keepdims=True)
        r = jnp.max(jnp.where(p == m, rev, -1.0), axis=0, keepdims=True)
        vals.append(m)
        lanes.append(r)
        p = jnp.where(rev == r, neg, p)
    top = jnp.concatenate(vals, axis=0)
    e = jnp.exp(top - top[:1])
    r8 = jnp.concatenate(lanes, axis=0)
    idx = jnp.int32(_N_EXPERTS - 1) - r8.astype(jnp.int32)
    wgt = e / jnp.sum(e, axis=0, keepdims=True)
    idx_ref[...] = idx.T
    wgt_ref[...] = wgt.T


@functools.partial(jax.jit, static_argnums=())
def kernel(hidden_states, weight):
    bsz, seq, h = hidden_states.shape
    tokens = bsz * seq
    x = hidden_states.reshape(tokens, h).astype(jnp.float32)
    w = weight.astype(jnp.float32)
    grid = (tokens // _TILE,)
    idx, wgt = pl.pallas_call(
        _gate_kernel,
        grid=grid,
        in_specs=[
            pl.BlockSpec((_TILE, h), lambda i: (i, 0)),
            pl.BlockSpec((_N_EXPERTS, h), lambda i: (0, 0)),
        ],
        out_specs=[
            pl.BlockSpec((_TILE, _TOP_K), lambda i: (i, 0)),
            pl.BlockSpec((_TILE, _TOP_K), lambda i: (i, 0)),
        ],
        out_shape=[
            jax.ShapeDtypeStruct((tokens, _TOP_K), jnp.int32),
            jax.ShapeDtypeStruct((tokens, _TOP_K), jnp.float32),
        ],
        compiler_params=pltpu.CompilerParams(
            dimension_semantics=("parallel",)
        ),
    )(x, w)
    return idx, wgt
